# baseline (speedup 1.0000x reference)
import functools

import jax
import jax.numpy as jnp
from jax import lax
from jax.experimental import pallas as pl
from jax.experimental.pallas import tpu as pltpu
from jax.experimental.pallas import tpu_sc as plsc

N = 10000
E = 320000
D = 128
NC, NS = 2, 16
NW = NC * NS
CH = 128
EW = 10368
PE = EW * NW
NCH = EW // CH
NSLOT = 3
NT = NCH // NSLOT
TCH = NW * NCH
C0_CH = 156
C1_CH = (TCH - NS * C0_CH) // NS
N_ACC = 10112
RPW = N_ACC // NS
BM = 1000

_mesh = plsc.VectorSubcoreMesh(
    core_axis_name="c", subcore_axis_name="s", num_cores=NC, num_subcores=NS)


def _fill(buf, nrows, value):
    v = jnp.full((16,), value, jnp.float32)

    @pl.loop(0, nrows)
    def _(r):
        @pl.loop(0, D, step=16)
        def _(cc):
            buf[r, pl.ds(cc, 16)] = v


def _zero_acc(tile, acc, r0):
    @pl.loop(0, RPW // CH)
    def _(k):
        pltpu.sync_copy(tile, acc.at[pl.ds(r0 + k * CH, CH)])

    rem = RPW % CH
    if rem:
        pltpu.sync_copy(tile.at[pl.ds(0, rem)],
                        acc.at[pl.ds(r0 + (RPW // CH) * CH, rem)])


@functools.partial(
    pl.kernel,
    out_type=jax.ShapeDtypeStruct((NC, N_ACC, D), jnp.float32),
    mesh=_mesh,
    scratch_types=[pltpu.VMEM((CH, D), jnp.float32),
                   pltpu.VMEM((NSLOT, CH), jnp.int32)]
      + [pltpu.SemaphoreType.DMA] * NSLOT
      + [pltpu.VMEM_SHARED((N_ACC, D), jnp.float32)],
)
def _sc_degree(dst_hbm, out_hbm, ones, didx, *rest):
    didxs = [didx.at[j] for j in range(NSLOT)]
    sems = rest[:NSLOT]
    acc = rest[NSLOT]
    core = lax.axis_index("c")
    sid = lax.axis_index("s")
    wid = sid * NC + core
    r0 = sid * RPW

    _fill(ones, CH, 0.0)
    _zero_acc(ones, acc, r0)
    _fill(ones, CH, 1.0)
    plsc.subcore_barrier()

    base = wid * NCH
    for j in range(NSLOT):
        pltpu.async_copy(dst_hbm.at[base + j], didxs[j], sems[j])

    @pl.loop(0, NT)
    def _(t):
        for j in range(NSLOT):
            pltpu.make_async_copy(dst_hbm.at[0], didxs[j], sems[j]).wait()
            pltpu.async_copy(ones, acc.at[didxs[j]], sems[j], add=True)
        for j in range(NSLOT):
            pltpu.make_async_copy(ones, acc.at[didxs[j]], sems[j]).wait()

            @pl.when(t < NT - 1)
            def _():
                pltpu.async_copy(dst_hbm.at[base + (t + 1) * NSLOT + j],
                                 didxs[j], sems[j])

    plsc.subcore_barrier()
    pltpu.sync_copy(acc.at[pl.ds(r0, RPW)], out_hbm.at[core, pl.ds(r0, RPW)])


@functools.partial(
    pl.kernel,
    out_type=jax.ShapeDtypeStruct((NC, N_ACC, D), jnp.float32),
    mesh=_mesh,
    scratch_types=[pltpu.VMEM((CH, D), jnp.float32)] * NSLOT
      + [pltpu.VMEM((NSLOT, CH), jnp.int32),
         pltpu.VMEM((NSLOT, CH), jnp.int32)]
      + [pltpu.SemaphoreType.DMA] * NSLOT
      + [pltpu.VMEM_SHARED((N_ACC, D), jnp.float32)],
)
def _sc_segsum(y_hbm, src_hbm, dst_hbm, out_hbm, *rest):
    rows = rest[:NSLOT]
    sidx, didx = rest[NSLOT], rest[NSLOT + 1]
    sidxs = [sidx.at[j] for j in range(NSLOT)]
    didxs = [didx.at[j] for j in range(NSLOT)]
    sems = rest[NSLOT + 2:2 * NSLOT + 2]
    acc = rest[2 * NSLOT + 2]
    core = lax.axis_index("c")
    sid = lax.axis_index("s")
    r0 = sid * RPW

    _fill(rows[0], CH, 0.0)
    _zero_acc(rows[0], acc, r0)
    plsc.subcore_barrier()

    base = jnp.where(core == 0, sid * C0_CH, NS * C0_CH + sid * C1_CH)
    my_nt = jnp.where(core == 0, C0_CH // NSLOT, C1_CH // NSLOT)

    def _issue_idx(j, c):
        pltpu.async_copy(src_hbm.at[base + c], sidxs[j], sems[j])
        pltpu.async_copy(dst_hbm.at[base + c], didxs[j], sems[j])

    def _wait_idx(j):
        pltpu.make_async_copy(src_hbm.at[0], sidxs[j], sems[j]).wait()
        pltpu.make_async_copy(src_hbm.at[0], didxs[j], sems[j]).wait()

    @pl.when(my_nt > 0)
    def _():
        for j in range(NSLOT):
            _issue_idx(j, j)

    @pl.loop(0, my_nt)
    def _(t):
        for j in range(NSLOT):
            _wait_idx(j)
            pltpu.async_copy(y_hbm.at[sidxs[j]], rows[j], sems[j])
        for j in range(NSLOT):
            pltpu.make_async_copy(y_hbm.at[sidxs[j]], rows[j],
                                  sems[j]).wait()
            pltpu.async_copy(rows[j], acc.at[didxs[j]], sems[j], add=True)
        for j in range(NSLOT):
            pltpu.make_async_copy(rows[j], acc.at[didxs[j]], sems[j]).wait()

            @pl.when(t < my_nt - 1)
            def _():
                _issue_idx(j, (t + 1) * NSLOT + j)

    plsc.subcore_barrier()
    pltpu.sync_copy(acc.at[pl.ds(r0, RPW)], out_hbm.at[core, pl.ds(r0, RPW)])


def _row_spec(i_map=lambda i: (i, 0)):
    return pl.BlockSpec((BM, D), i_map)


def _mm_body(x_ref, w_ref, o_ref):
    o_ref[...] = jnp.dot(x_ref[...], w_ref[...],
                         preferred_element_type=jnp.float32)


def _tc_mm(x, w):
    return pl.pallas_call(
        _mm_body,
        grid=(N // BM,),
        in_specs=[_row_spec(), pl.BlockSpec((D, D), lambda i: (0, 0))],
        out_specs=_row_spec(),
        out_shape=jax.ShapeDtypeStruct((N, D), jnp.float32),
    )(x, w)


def _scale_body(degp_ref, xw_ref, dis_ref, y_ref):
    deg = 1.0 + degp_ref[0] + degp_ref[1]
    dis = lax.rsqrt(deg)
    dis_ref[...] = dis
    y_ref[...] = dis * xw_ref[...]


def _tc_scale(degp, xw):
    return pl.pallas_call(
        _scale_body,
        grid=(N // BM,),
        in_specs=[pl.BlockSpec((NC, BM, D), lambda i: (0, i, 0)), _row_spec()],
        out_specs=[_row_spec(), _row_spec()],
        out_shape=[jax.ShapeDtypeStruct((N, D), jnp.float32),
                   jax.ShapeDtypeStruct((N, D), jnp.float32)],
    )(degp, xw)


def _l2_body(p_ref, y1_ref, dis_ref, b1_ref, w2_ref, o_ref):
    s = p_ref[0] + p_ref[1] + y1_ref[...]
    h = jnp.maximum(dis_ref[...] * s + b1_ref[...], 0.0)
    o_ref[...] = dis_ref[...] * jnp.dot(h, w2_ref[...],
                                        preferred_element_type=jnp.float32)


def _tc_layer2(p1, y1, dis, b1r, w2p):
    return pl.pallas_call(
        _l2_body,
        grid=(N // BM,),
        in_specs=[pl.BlockSpec((NC, BM, D), lambda i: (0, i, 0)),
                  _row_spec(), _row_spec(),
                  pl.BlockSpec((1, D), lambda i: (0, 0)),
                  pl.BlockSpec((D, D), lambda i: (0, 0))],
        out_specs=_row_spec(),
        out_shape=jax.ShapeDtypeStruct((N, D), jnp.float32),
    )(p1, y1, dis, b1r, w2p)


def _fin_body(p_ref, y2_ref, dis_ref, b2_ref, o_ref):
    z = dis_ref[...] * (p_ref[0] + p_ref[1] + y2_ref[...]) + b2_ref[...]
    col = lax.broadcasted_iota(jnp.int32, z.shape, 1)
    valid = col < 64
    zm = jnp.where(valid, z, -jnp.inf)
    m = jnp.max(zm, axis=1, keepdims=True)
    e = jnp.where(valid, jnp.exp(z - m), 0.0)
    ssum = jnp.sum(e, axis=1, keepdims=True)
    o_ref[...] = z - m - jnp.log(ssum)


def _tc_final(p2, y2, dis, b2r):
    return pl.pallas_call(
        _fin_body,
        grid=(N // BM,),
        in_specs=[pl.BlockSpec((NC, BM, D), lambda i: (0, i, 0)),
                  _row_spec(), _row_spec(),
                  pl.BlockSpec((1, D), lambda i: (0, 0))],
        out_specs=_row_spec(),
        out_shape=jax.ShapeDtypeStruct((N, D), jnp.float32),
    )(p2, y2, dis, b2r)


def kernel(x, edge_index, W1, b1, W2, b2):
    src = edge_index[0].astype(jnp.int32)
    dst = edge_index[1].astype(jnp.int32)
    src = jnp.concatenate([src, jnp.zeros((PE - E,), jnp.int32)])
    dst = jnp.concatenate([dst, jnp.full((PE - E,), N, jnp.int32)])
    src = src.reshape(NW * NCH, CH)
    dst = dst.reshape(NW * NCH, CH)

    w2p = jnp.pad(W2, ((0, 0), (0, D - W2.shape[1])))
    b1r = b1.reshape(1, D)
    b2r = jnp.pad(b2, (0, D - b2.shape[0])).reshape(1, D)

    degp = _sc_degree(dst)
    xw = _tc_mm(x, W1)
    dis, y1 = _tc_scale(degp, xw)
    p1 = _sc_segsum(y1, src, dst)
    y2 = _tc_layer2(p1, y1, dis, b1r, w2p)
    p2 = _sc_segsum(y2, src, dst)
    outp = _tc_final(p2, y2, dis, b2r)
    return outp[:, :64]


# speedup vs baseline: 1.0021x; 1.0021x over previous
import functools

import jax
import jax.numpy as jnp
from jax import lax
from jax.experimental import pallas as pl
from jax.experimental.pallas import tpu as pltpu
from jax.experimental.pallas import tpu_sc as plsc

N = 10000
E = 320000
D = 128
NC, NS = 2, 16
NW = NC * NS
CH = 128
EW = 10368
PE = EW * NW
NCH = EW // CH
NSLOT = 3
NT = NCH // NSLOT
TCH = NW * NCH
C0_CH = 150
C1_CH = (TCH - NS * C0_CH) // NS
N_ACC = 10112
RPW = N_ACC // NS
BM = 1000

_mesh = plsc.VectorSubcoreMesh(
    core_axis_name="c", subcore_axis_name="s", num_cores=NC, num_subcores=NS)


def _fill(buf, nrows, value):
    v = jnp.full((16,), value, jnp.float32)

    @pl.loop(0, nrows)
    def _(r):
        @pl.loop(0, D, step=16)
        def _(cc):
            buf[r, pl.ds(cc, 16)] = v


def _zero_acc(tile, acc, r0):
    @pl.loop(0, RPW // CH)
    def _(k):
        pltpu.sync_copy(tile, acc.at[pl.ds(r0 + k * CH, CH)])

    rem = RPW % CH
    if rem:
        pltpu.sync_copy(tile.at[pl.ds(0, rem)],
                        acc.at[pl.ds(r0 + (RPW // CH) * CH, rem)])


@functools.partial(
    pl.kernel,
    out_type=jax.ShapeDtypeStruct((NC, N_ACC, D), jnp.float32),
    mesh=_mesh,
    scratch_types=[pltpu.VMEM((CH, D), jnp.float32),
                   pltpu.VMEM((NSLOT, CH), jnp.int32)]
      + [pltpu.SemaphoreType.DMA] * NSLOT
      + [pltpu.VMEM_SHARED((N_ACC, D), jnp.float32)],
)
def _sc_degree(dst_hbm, out_hbm, ones, didx, *rest):
    didxs = [didx.at[j] for j in range(NSLOT)]
    sems = rest[:NSLOT]
    acc = rest[NSLOT]
    core = lax.axis_index("c")
    sid = lax.axis_index("s")
    wid = sid * NC + core
    r0 = sid * RPW

    _fill(ones, CH, 0.0)
    _zero_acc(ones, acc, r0)
    _fill(ones, CH, 1.0)
    plsc.subcore_barrier()

    base = wid * NCH
    for j in range(NSLOT):
        pltpu.async_copy(dst_hbm.at[base + j], didxs[j], sems[j])

    @pl.loop(0, NT)
    def _(t):
        for j in range(NSLOT):
            pltpu.make_async_copy(dst_hbm.at[0], didxs[j], sems[j]).wait()
            pltpu.async_copy(ones, acc.at[didxs[j]], sems[j], add=True)
        for j in range(NSLOT):
            pltpu.make_async_copy(ones, acc.at[didxs[j]], sems[j]).wait()

            @pl.when(t < NT - 1)
            def _():
                pltpu.async_copy(dst_hbm.at[base + (t + 1) * NSLOT + j],
                                 didxs[j], sems[j])

    plsc.subcore_barrier()
    pltpu.sync_copy(acc.at[pl.ds(r0, RPW)], out_hbm.at[core, pl.ds(r0, RPW)])


@functools.partial(
    pl.kernel,
    out_type=jax.ShapeDtypeStruct((NC, N_ACC, D), jnp.float32),
    mesh=_mesh,
    scratch_types=[pltpu.VMEM((CH, D), jnp.float32)] * NSLOT
      + [pltpu.VMEM((NSLOT, CH), jnp.int32),
         pltpu.VMEM((NSLOT, CH), jnp.int32)]
      + [pltpu.SemaphoreType.DMA] * NSLOT
      + [pltpu.VMEM_SHARED((N_ACC, D), jnp.float32)],
)
def _sc_segsum(y_hbm, src_hbm, dst_hbm, out_hbm, *rest):
    rows = rest[:NSLOT]
    sidx, didx = rest[NSLOT], rest[NSLOT + 1]
    sidxs = [sidx.at[j] for j in range(NSLOT)]
    didxs = [didx.at[j] for j in range(NSLOT)]
    sems = rest[NSLOT + 2:2 * NSLOT + 2]
    acc = rest[2 * NSLOT + 2]
    core = lax.axis_index("c")
    sid = lax.axis_index("s")
    r0 = sid * RPW

    _fill(rows[0], CH, 0.0)
    _zero_acc(rows[0], acc, r0)
    plsc.subcore_barrier()

    base = jnp.where(core == 0, sid * C0_CH, NS * C0_CH + sid * C1_CH)
    my_nt = jnp.where(core == 0, C0_CH // NSLOT, C1_CH // NSLOT)

    def _issue_idx(j, c):
        pltpu.async_copy(src_hbm.at[base + c], sidxs[j], sems[j])
        pltpu.async_copy(dst_hbm.at[base + c], didxs[j], sems[j])

    def _wait_idx(j):
        pltpu.make_async_copy(src_hbm.at[0], sidxs[j], sems[j]).wait()
        pltpu.make_async_copy(src_hbm.at[0], didxs[j], sems[j]).wait()

    @pl.when(my_nt > 0)
    def _():
        for j in range(NSLOT):
            _issue_idx(j, j)

    @pl.loop(0, my_nt)
    def _(t):
        for j in range(NSLOT):
            _wait_idx(j)
            pltpu.async_copy(y_hbm.at[sidxs[j]], rows[j], sems[j])
        for j in range(NSLOT):
            pltpu.make_async_copy(y_hbm.at[sidxs[j]], rows[j],
                                  sems[j]).wait()
            pltpu.async_copy(rows[j], acc.at[didxs[j]], sems[j], add=True)
        for j in range(NSLOT):
            pltpu.make_async_copy(rows[j], acc.at[didxs[j]], sems[j]).wait()

            @pl.when(t < my_nt - 1)
            def _():
                _issue_idx(j, (t + 1) * NSLOT + j)

    plsc.subcore_barrier()
    pltpu.sync_copy(acc.at[pl.ds(r0, RPW)], out_hbm.at[core, pl.ds(r0, RPW)])


def _row_spec(i_map=lambda i: (i, 0)):
    return pl.BlockSpec((BM, D), i_map)


def _mm_body(x_ref, w_ref, o_ref):
    o_ref[...] = jnp.dot(x_ref[...], w_ref[...],
                         preferred_element_type=jnp.float32)


def _tc_mm(x, w):
    return pl.pallas_call(
        _mm_body,
        grid=(N // BM,),
        in_specs=[_row_spec(), pl.BlockSpec((D, D), lambda i: (0, 0))],
        out_specs=_row_spec(),
        out_shape=jax.ShapeDtypeStruct((N, D), jnp.float32),
    )(x, w)


def _scale_body(degp_ref, xw_ref, dis_ref, y_ref):
    deg = 1.0 + degp_ref[0] + degp_ref[1]
    dis = lax.rsqrt(deg)
    dis_ref[...] = dis
    y_ref[...] = dis * xw_ref[...]


def _tc_scale(degp, xw):
    return pl.pallas_call(
        _scale_body,
        grid=(N // BM,),
        in_specs=[pl.BlockSpec((NC, BM, D), lambda i: (0, i, 0)), _row_spec()],
        out_specs=[_row_spec(), _row_spec()],
        out_shape=[jax.ShapeDtypeStruct((N, D), jnp.float32),
                   jax.ShapeDtypeStruct((N, D), jnp.float32)],
    )(degp, xw)


def _l2_body(p_ref, y1_ref, dis_ref, b1_ref, w2_ref, o_ref):
    s = p_ref[0] + p_ref[1] + y1_ref[...]
    h = jnp.maximum(dis_ref[...] * s + b1_ref[...], 0.0)
    o_ref[...] = dis_ref[...] * jnp.dot(h, w2_ref[...],
                                        preferred_element_type=jnp.float32)


def _tc_layer2(p1, y1, dis, b1r, w2p):
    return pl.pallas_call(
        _l2_body,
        grid=(N // BM,),
        in_specs=[pl.BlockSpec((NC, BM, D), lambda i: (0, i, 0)),
                  _row_spec(), _row_spec(),
                  pl.BlockSpec((1, D), lambda i: (0, 0)),
                  pl.BlockSpec((D, D), lambda i: (0, 0))],
        out_specs=_row_spec(),
        out_shape=jax.ShapeDtypeStruct((N, D), jnp.float32),
    )(p1, y1, dis, b1r, w2p)


def _fin_body(p_ref, y2_ref, dis_ref, b2_ref, o_ref):
    z = dis_ref[...] * (p_ref[0] + p_ref[1] + y2_ref[...]) + b2_ref[...]
    col = lax.broadcasted_iota(jnp.int32, z.shape, 1)
    valid = col < 64
    zm = jnp.where(valid, z, -jnp.inf)
    m = jnp.max(zm, axis=1, keepdims=True)
    e = jnp.where(valid, jnp.exp(z - m), 0.0)
    ssum = jnp.sum(e, axis=1, keepdims=True)
    o_ref[...] = z - m - jnp.log(ssum)


def _tc_final(p2, y2, dis, b2r):
    return pl.pallas_call(
        _fin_body,
        grid=(N // BM,),
        in_specs=[pl.BlockSpec((NC, BM, D), lambda i: (0, i, 0)),
                  _row_spec(), _row_spec(),
                  pl.BlockSpec((1, D), lambda i: (0, 0))],
        out_specs=_row_spec(),
        out_shape=jax.ShapeDtypeStruct((N, D), jnp.float32),
    )(p2, y2, dis, b2r)


def kernel(x, edge_index, W1, b1, W2, b2):
    src = edge_index[0].astype(jnp.int32)
    dst = edge_index[1].astype(jnp.int32)
    src = jnp.concatenate([src, jnp.zeros((PE - E,), jnp.int32)])
    dst = jnp.concatenate([dst, jnp.full((PE - E,), N, jnp.int32)])
    src = src.reshape(NW * NCH, CH)
    dst = dst.reshape(NW * NCH, CH)

    w2p = jnp.pad(W2, ((0, 0), (0, D - W2.shape[1])))
    b1r = b1.reshape(1, D)
    b2r = jnp.pad(b2, (0, D - b2.shape[0])).reshape(1, D)

    degp = _sc_degree(dst)
    xw = _tc_mm(x, W1)
    dis, y1 = _tc_scale(degp, xw)
    p1 = _sc_segsum(y1, src, dst)
    y2 = _tc_layer2(p1, y1, dis, b1r, w2p)
    p2 = _sc_segsum(y2, src, dst)
    outp = _tc_final(p2, y2, dis, b2r)
    return outp[:, :64]


# speedup vs baseline: 1.0024x; 1.0003x over previous
import functools

import jax
import jax.numpy as jnp
from jax import lax
from jax.experimental import pallas as pl
from jax.experimental.pallas import tpu as pltpu
from jax.experimental.pallas import tpu_sc as plsc

N = 10000
E = 320000
D = 128
NC, NS = 2, 16
NW = NC * NS
CH = 128
EW = 10368
PE = EW * NW
NCH = EW // CH
NSLOT = 3
NT = NCH // NSLOT
TCH = NW * NCH
C0_CH = 150
C1_CH = (TCH - NS * C0_CH) // NS
N_ACC = 10112
RPW = N_ACC // NS
BM = 1000

_mesh = plsc.VectorSubcoreMesh(
    core_axis_name="c", subcore_axis_name="s", num_cores=NC, num_subcores=NS)


def _fill(buf, nrows, value):
    v = jnp.full((16,), value, jnp.float32)

    @pl.loop(0, nrows)
    def _(r):
        @pl.loop(0, D, step=16)
        def _(cc):
            buf[r, pl.ds(cc, 16)] = v


def _zero_acc(tile, acc, r0):
    @pl.loop(0, RPW // CH)
    def _(k):
        pltpu.sync_copy(tile, acc.at[pl.ds(r0 + k * CH, CH)])

    rem = RPW % CH
    if rem:
        pltpu.sync_copy(tile.at[pl.ds(0, rem)],
                        acc.at[pl.ds(r0 + (RPW // CH) * CH, rem)])


@functools.partial(
    pl.kernel,
    out_type=jax.ShapeDtypeStruct((NC, N_ACC, D), jnp.float32),
    mesh=_mesh,
    scratch_types=[pltpu.VMEM((CH, D), jnp.float32),
                   pltpu.VMEM((NSLOT, CH), jnp.int32)]
      + [pltpu.SemaphoreType.DMA] * NSLOT
      + [pltpu.VMEM_SHARED((N_ACC, D), jnp.float32)],
)
def _sc_degree(dst_hbm, out_hbm, ones, didx, *rest):
    didxs = [didx.at[j] for j in range(NSLOT)]
    sems = rest[:NSLOT]
    acc = rest[NSLOT]
    core = lax.axis_index("c")
    sid = lax.axis_index("s")
    wid = sid * NC + core
    r0 = sid * RPW

    _fill(ones, CH, 0.0)
    _zero_acc(ones, acc, r0)
    _fill(ones, CH, 1.0)
    plsc.subcore_barrier()

    base = wid * NCH
    for j in range(NSLOT):
        pltpu.async_copy(dst_hbm.at[base + j], didxs[j], sems[j])

    @pl.loop(0, NT)
    def _(t):
        for j in range(NSLOT):
            pltpu.make_async_copy(dst_hbm.at[0], didxs[j], sems[j]).wait()
            pltpu.async_copy(ones, acc.at[didxs[j]], sems[j], add=True)
        for j in range(NSLOT):
            pltpu.make_async_copy(ones, acc.at[didxs[j]], sems[j]).wait()

            @pl.when(t < NT - 1)
            def _():
                pltpu.async_copy(dst_hbm.at[base + (t + 1) * NSLOT + j],
                                 didxs[j], sems[j])

    plsc.subcore_barrier()
    pltpu.sync_copy(acc.at[pl.ds(r0, RPW)], out_hbm.at[core, pl.ds(r0, RPW)])


@functools.partial(
    pl.kernel,
    out_type=jax.ShapeDtypeStruct((NC, N_ACC, D), jnp.float32),
    mesh=_mesh,
    scratch_types=[pltpu.VMEM((CH, D), jnp.float32)] * NSLOT
      + [pltpu.VMEM((NSLOT, CH), jnp.int32),
         pltpu.VMEM((NSLOT, CH), jnp.int32)]
      + [pltpu.SemaphoreType.DMA] * (2 * NSLOT)
      + [pltpu.VMEM_SHARED((N_ACC, D), jnp.float32)],
)
def _sc_segsum(y_hbm, src_hbm, dst_hbm, out_hbm, *rest):
    rows = rest[:NSLOT]
    sidx, didx = rest[NSLOT], rest[NSLOT + 1]
    didxs = [didx.at[j] for j in range(NSLOT)]
    semsa = rest[NSLOT + 2:2 * NSLOT + 2]
    semsb = rest[2 * NSLOT + 2:3 * NSLOT + 2]
    acc = rest[3 * NSLOT + 2]
    H = CH // 2
    core = lax.axis_index("c")
    sid = lax.axis_index("s")
    r0 = sid * RPW

    _fill(rows[0], CH, 0.0)
    _zero_acc(rows[0], acc, r0)
    plsc.subcore_barrier()

    base = jnp.where(core == 0, sid * C0_CH, NS * C0_CH + sid * C1_CH)
    my_nt = jnp.where(core == 0, C0_CH // NSLOT, C1_CH // NSLOT)

    def _issue_idx(j, c):
        pltpu.async_copy(src_hbm.at[base + c], sidx.at[j], semsb[j])
        pltpu.async_copy(dst_hbm.at[base + c], didxs[j], semsb[j])

    def _wait_idx(j):
        pltpu.make_async_copy(src_hbm.at[0], sidx.at[j], semsb[j]).wait()
        pltpu.make_async_copy(src_hbm.at[0], didxs[j], semsb[j]).wait()

    def _issue_gather(j):
        pltpu.async_copy(y_hbm.at[sidx.at[j, pl.ds(0, H)]],
                         rows[j].at[pl.ds(0, H)], semsa[j])
        pltpu.async_copy(y_hbm.at[sidx.at[j, pl.ds(H, H)]],
                         rows[j].at[pl.ds(H, H)], semsb[j])

    def _wait_gather(j):
        pltpu.make_async_copy(y_hbm.at[sidx.at[j, pl.ds(0, H)]],
                              rows[j].at[pl.ds(0, H)], semsa[j]).wait()
        pltpu.make_async_copy(y_hbm.at[sidx.at[j, pl.ds(0, H)]],
                              rows[j].at[pl.ds(H, H)], semsb[j]).wait()

    @pl.when(my_nt > 0)
    def _():
        for j in range(NSLOT):
            _issue_idx(j, j)

    @pl.loop(0, my_nt)
    def _(t):
        for j in range(NSLOT):
            _wait_idx(j)
            _issue_gather(j)
        for j in range(NSLOT):
            _wait_gather(j)
            pltpu.async_copy(rows[j], acc.at[didxs[j]], semsa[j], add=True)
        for j in range(NSLOT):
            pltpu.make_async_copy(rows[j], acc.at[didxs[j]], semsa[j]).wait()

            @pl.when(t < my_nt - 1)
            def _():
                _issue_idx(j, (t + 1) * NSLOT + j)

    plsc.subcore_barrier()
    pltpu.sync_copy(acc.at[pl.ds(r0, RPW)], out_hbm.at[core, pl.ds(r0, RPW)])


def _row_spec(i_map=lambda i: (i, 0)):
    return pl.BlockSpec((BM, D), i_map)


def _mm_body(x_ref, w_ref, o_ref):
    o_ref[...] = jnp.dot(x_ref[...], w_ref[...],
                         preferred_element_type=jnp.float32)


def _tc_mm(x, w):
    return pl.pallas_call(
        _mm_body,
        grid=(N // BM,),
        in_specs=[_row_spec(), pl.BlockSpec((D, D), lambda i: (0, 0))],
        out_specs=_row_spec(),
        out_shape=jax.ShapeDtypeStruct((N, D), jnp.float32),
    )(x, w)


def _scale_body(degp_ref, xw_ref, dis_ref, y_ref):
    deg = 1.0 + degp_ref[0] + degp_ref[1]
    dis = lax.rsqrt(deg)
    dis_ref[...] = dis
    y_ref[...] = dis * xw_ref[...]


def _tc_scale(degp, xw):
    return pl.pallas_call(
        _scale_body,
        grid=(N // BM,),
        in_specs=[pl.BlockSpec((NC, BM, D), lambda i: (0, i, 0)), _row_spec()],
        out_specs=[_row_spec(), _row_spec()],
        out_shape=[jax.ShapeDtypeStruct((N, D), jnp.float32),
                   jax.ShapeDtypeStruct((N, D), jnp.float32)],
    )(degp, xw)


def _l2_body(p_ref, y1_ref, dis_ref, b1_ref, w2_ref, o_ref):
    s = p_ref[0] + p_ref[1] + y1_ref[...]
    h = jnp.maximum(dis_ref[...] * s + b1_ref[...], 0.0)
    o_ref[...] = dis_ref[...] * jnp.dot(h, w2_ref[...],
                                        preferred_element_type=jnp.float32)


def _tc_layer2(p1, y1, dis, b1r, w2p):
    return pl.pallas_call(
        _l2_body,
        grid=(N // BM,),
        in_specs=[pl.BlockSpec((NC, BM, D), lambda i: (0, i, 0)),
                  _row_spec(), _row_spec(),
                  pl.BlockSpec((1, D), lambda i: (0, 0)),
                  pl.BlockSpec((D, D), lambda i: (0, 0))],
        out_specs=_row_spec(),
        out_shape=jax.ShapeDtypeStruct((N, D), jnp.float32),
    )(p1, y1, dis, b1r, w2p)


def _fin_body(p_ref, y2_ref, dis_ref, b2_ref, o_ref):
    z = dis_ref[...] * (p_ref[0] + p_ref[1] + y2_ref[...]) + b2_ref[...]
    col = lax.broadcasted_iota(jnp.int32, z.shape, 1)
    valid = col < 64
    zm = jnp.where(valid, z, -jnp.inf)
    m = jnp.max(zm, axis=1, keepdims=True)
    e = jnp.where(valid, jnp.exp(z - m), 0.0)
    ssum = jnp.sum(e, axis=1, keepdims=True)
    o_ref[...] = z - m - jnp.log(ssum)


def _tc_final(p2, y2, dis, b2r):
    return pl.pallas_call(
        _fin_body,
        grid=(N // BM,),
        in_specs=[pl.BlockSpec((NC, BM, D), lambda i: (0, i, 0)),
                  _row_spec(), _row_spec(),
                  pl.BlockSpec((1, D), lambda i: (0, 0))],
        out_specs=_row_spec(),
        out_shape=jax.ShapeDtypeStruct((N, D), jnp.float32),
    )(p2, y2, dis, b2r)


def kernel(x, edge_index, W1, b1, W2, b2):
    src = edge_index[0].astype(jnp.int32)
    dst = edge_index[1].astype(jnp.int32)
    src = jnp.concatenate([src, jnp.zeros((PE - E,), jnp.int32)])
    dst = jnp.concatenate([dst, jnp.full((PE - E,), N, jnp.int32)])
    src = src.reshape(NW * NCH, CH)
    dst = dst.reshape(NW * NCH, CH)

    w2p = jnp.pad(W2, ((0, 0), (0, D - W2.shape[1])))
    b1r = b1.reshape(1, D)
    b2r = jnp.pad(b2, (0, D - b2.shape[0])).reshape(1, D)

    degp = _sc_degree(dst)
    xw = _tc_mm(x, W1)
    dis, y1 = _tc_scale(degp, xw)
    p1 = _sc_segsum(y1, src, dst)
    y2 = _tc_layer2(p1, y1, dis, b1r, w2p)
    p2 = _sc_segsum(y2, src, dst)
    outp = _tc_final(p2, y2, dis, b2r)
    return outp[:, :64]


# speedup vs baseline: 1.0034x; 1.0010x over previous
import functools

import jax
import jax.numpy as jnp
from jax import lax
from jax.experimental import pallas as pl
from jax.experimental.pallas import tpu as pltpu
from jax.experimental.pallas import tpu_sc as plsc

N = 10000
E = 320000
D = 128
NC, NS = 2, 16
NW = NC * NS
CH = 128
EW = 10368
PE = EW * NW
NCH = EW // CH
NSLOT = 3
NT = NCH // NSLOT
TCH = NW * NCH
C0_CH = 147
C1_CH = (TCH - NS * C0_CH) // NS
N_ACC = 10112
RPW = N_ACC // NS
BM = 1000

_mesh = plsc.VectorSubcoreMesh(
    core_axis_name="c", subcore_axis_name="s", num_cores=NC, num_subcores=NS)


def _fill(buf, nrows, value):
    v = jnp.full((16,), value, jnp.float32)

    @pl.loop(0, nrows)
    def _(r):
        @pl.loop(0, D, step=16)
        def _(cc):
            buf[r, pl.ds(cc, 16)] = v


def _zero_acc(tile, acc, r0):
    @pl.loop(0, RPW // CH)
    def _(k):
        pltpu.sync_copy(tile, acc.at[pl.ds(r0 + k * CH, CH)])

    rem = RPW % CH
    if rem:
        pltpu.sync_copy(tile.at[pl.ds(0, rem)],
                        acc.at[pl.ds(r0 + (RPW // CH) * CH, rem)])


@functools.partial(
    pl.kernel,
    out_type=jax.ShapeDtypeStruct((NC, N_ACC, D), jnp.float32),
    mesh=_mesh,
    scratch_types=[pltpu.VMEM((CH, D), jnp.float32),
                   pltpu.VMEM((NSLOT, CH), jnp.int32)]
      + [pltpu.SemaphoreType.DMA] * NSLOT
      + [pltpu.VMEM_SHARED((N_ACC, D), jnp.float32)],
)
def _sc_degree(dst_hbm, out_hbm, ones, didx, *rest):
    didxs = [didx.at[j] for j in range(NSLOT)]
    sems = rest[:NSLOT]
    acc = rest[NSLOT]
    core = lax.axis_index("c")
    sid = lax.axis_index("s")
    wid = sid * NC + core
    r0 = sid * RPW

    _fill(ones, CH, 0.0)
    _zero_acc(ones, acc, r0)
    _fill(ones, CH, 1.0)
    plsc.subcore_barrier()

    base = wid * NCH
    for j in range(NSLOT):
        pltpu.async_copy(dst_hbm.at[base + j], didxs[j], sems[j])

    @pl.loop(0, NT)
    def _(t):
        for j in range(NSLOT):
            pltpu.make_async_copy(dst_hbm.at[0], didxs[j], sems[j]).wait()
            pltpu.async_copy(ones, acc.at[didxs[j]], sems[j], add=True)
        for j in range(NSLOT):
            pltpu.make_async_copy(ones, acc.at[didxs[j]], sems[j]).wait()

            @pl.when(t < NT - 1)
            def _():
                pltpu.async_copy(dst_hbm.at[base + (t + 1) * NSLOT + j],
                                 didxs[j], sems[j])

    plsc.subcore_barrier()
    pltpu.sync_copy(acc.at[pl.ds(r0, RPW)], out_hbm.at[core, pl.ds(r0, RPW)])


@functools.partial(
    pl.kernel,
    out_type=jax.ShapeDtypeStruct((NC, N_ACC, D), jnp.float32),
    mesh=_mesh,
    scratch_types=[pltpu.VMEM((CH, D), jnp.float32)] * NSLOT
      + [pltpu.VMEM((NSLOT, CH), jnp.int32),
         pltpu.VMEM((NSLOT, CH), jnp.int32)]
      + [pltpu.SemaphoreType.DMA] * (2 * NSLOT)
      + [pltpu.VMEM_SHARED((N_ACC, D), jnp.float32)],
)
def _sc_segsum(y_hbm, src_hbm, dst_hbm, out_hbm, *rest):
    rows = rest[:NSLOT]
    sidx, didx = rest[NSLOT], rest[NSLOT + 1]
    didxs = [didx.at[j] for j in range(NSLOT)]
    semsa = rest[NSLOT + 2:2 * NSLOT + 2]
    semsb = rest[2 * NSLOT + 2:3 * NSLOT + 2]
    acc = rest[3 * NSLOT + 2]
    H = CH // 2
    core = lax.axis_index("c")
    sid = lax.axis_index("s")
    r0 = sid * RPW

    _fill(rows[0], CH, 0.0)
    _zero_acc(rows[0], acc, r0)
    plsc.subcore_barrier()

    base = jnp.where(core == 0, sid * C0_CH, NS * C0_CH + sid * C1_CH)
    my_nt = jnp.where(core == 0, C0_CH // NSLOT, C1_CH // NSLOT)

    def _issue_idx(j, c):
        pltpu.async_copy(src_hbm.at[base + c], sidx.at[j], semsb[j])
        pltpu.async_copy(dst_hbm.at[base + c], didxs[j], semsb[j])

    def _wait_idx(j):
        pltpu.make_async_copy(src_hbm.at[0], sidx.at[j], semsb[j]).wait()
        pltpu.make_async_copy(src_hbm.at[0], didxs[j], semsb[j]).wait()

    def _issue_gather(j):
        pltpu.async_copy(y_hbm.at[sidx.at[j, pl.ds(0, H)]],
                         rows[j].at[pl.ds(0, H)], semsa[j])
        pltpu.async_copy(y_hbm.at[sidx.at[j, pl.ds(H, H)]],
                         rows[j].at[pl.ds(H, H)], semsb[j])

    def _wait_gather(j):
        pltpu.make_async_copy(y_hbm.at[sidx.at[j, pl.ds(0, H)]],
                              rows[j].at[pl.ds(0, H)], semsa[j]).wait()
        pltpu.make_async_copy(y_hbm.at[sidx.at[j, pl.ds(0, H)]],
                              rows[j].at[pl.ds(H, H)], semsb[j]).wait()

    @pl.when(my_nt > 0)
    def _():
        for j in range(NSLOT):
            _issue_idx(j, j)

    @pl.loop(0, my_nt)
    def _(t):
        for j in range(NSLOT):
            _wait_idx(j)
            _issue_gather(j)
        for j in range(NSLOT):
            _wait_gather(j)
            pltpu.async_copy(rows[j], acc.at[didxs[j]], semsa[j], add=True)
        for j in range(NSLOT):
            pltpu.make_async_copy(rows[j], acc.at[didxs[j]], semsa[j]).wait()

            @pl.when(t < my_nt - 1)
            def _():
                _issue_idx(j, (t + 1) * NSLOT + j)

    plsc.subcore_barrier()
    pltpu.sync_copy(acc.at[pl.ds(r0, RPW)], out_hbm.at[core, pl.ds(r0, RPW)])


def _row_spec(i_map=lambda i: (i, 0)):
    return pl.BlockSpec((BM, D), i_map)


def _mm_body(x_ref, w_ref, o_ref):
    o_ref[...] = jnp.dot(x_ref[...], w_ref[...],
                         preferred_element_type=jnp.float32)


def _tc_mm(x, w):
    return pl.pallas_call(
        _mm_body,
        grid=(N // BM,),
        in_specs=[_row_spec(), pl.BlockSpec((D, D), lambda i: (0, 0))],
        out_specs=_row_spec(),
        out_shape=jax.ShapeDtypeStruct((N, D), jnp.float32),
    )(x, w)


def _scale_body(degp_ref, xw_ref, dis_ref, y_ref):
    deg = 1.0 + degp_ref[0] + degp_ref[1]
    dis = lax.rsqrt(deg)
    dis_ref[...] = dis
    y_ref[...] = dis * xw_ref[...]


def _tc_scale(degp, xw):
    return pl.pallas_call(
        _scale_body,
        grid=(N // BM,),
        in_specs=[pl.BlockSpec((NC, BM, D), lambda i: (0, i, 0)), _row_spec()],
        out_specs=[_row_spec(), _row_spec()],
        out_shape=[jax.ShapeDtypeStruct((N, D), jnp.float32),
                   jax.ShapeDtypeStruct((N, D), jnp.float32)],
    )(degp, xw)


def _l2_body(p_ref, y1_ref, dis_ref, b1_ref, w2_ref, o_ref):
    s = p_ref[0] + p_ref[1] + y1_ref[...]
    h = jnp.maximum(dis_ref[...] * s + b1_ref[...], 0.0)
    o_ref[...] = dis_ref[...] * jnp.dot(h, w2_ref[...],
                                        preferred_element_type=jnp.float32)


def _tc_layer2(p1, y1, dis, b1r, w2p):
    return pl.pallas_call(
        _l2_body,
        grid=(N // BM,),
        in_specs=[pl.BlockSpec((NC, BM, D), lambda i: (0, i, 0)),
                  _row_spec(), _row_spec(),
                  pl.BlockSpec((1, D), lambda i: (0, 0)),
                  pl.BlockSpec((D, D), lambda i: (0, 0))],
        out_specs=_row_spec(),
        out_shape=jax.ShapeDtypeStruct((N, D), jnp.float32),
    )(p1, y1, dis, b1r, w2p)


def _fin_body(p_ref, y2_ref, dis_ref, b2_ref, o_ref):
    z = dis_ref[...] * (p_ref[0] + p_ref[1] + y2_ref[...]) + b2_ref[...]
    col = lax.broadcasted_iota(jnp.int32, z.shape, 1)
    valid = col < 64
    zm = jnp.where(valid, z, -jnp.inf)
    m = jnp.max(zm, axis=1, keepdims=True)
    e = jnp.where(valid, jnp.exp(z - m), 0.0)
    ssum = jnp.sum(e, axis=1, keepdims=True)
    o_ref[...] = z - m - jnp.log(ssum)


def _tc_final(p2, y2, dis, b2r):
    return pl.pallas_call(
        _fin_body,
        grid=(N // BM,),
        in_specs=[pl.BlockSpec((NC, BM, D), lambda i: (0, i, 0)),
                  _row_spec(), _row_spec(),
                  pl.BlockSpec((1, D), lambda i: (0, 0))],
        out_specs=_row_spec(),
        out_shape=jax.ShapeDtypeStruct((N, D), jnp.float32),
    )(p2, y2, dis, b2r)


def kernel(x, edge_index, W1, b1, W2, b2):
    src = edge_index[0].astype(jnp.int32)
    dst = edge_index[1].astype(jnp.int32)
    src = jnp.concatenate([src, jnp.zeros((PE - E,), jnp.int32)])
    dst = jnp.concatenate([dst, jnp.full((PE - E,), N, jnp.int32)])
    src = src.reshape(NW * NCH, CH)
    dst = dst.reshape(NW * NCH, CH)

    w2p = jnp.pad(W2, ((0, 0), (0, D - W2.shape[1])))
    b1r = b1.reshape(1, D)
    b2r = jnp.pad(b2, (0, D - b2.shape[0])).reshape(1, D)

    degp = _sc_degree(dst)
    xw = _tc_mm(x, W1)
    dis, y1 = _tc_scale(degp, xw)
    p1 = _sc_segsum(y1, src, dst)
    y2 = _tc_layer2(p1, y1, dis, b1r, w2p)
    p2 = _sc_segsum(y2, src, dst)
    outp = _tc_final(p2, y2, dis, b2r)
    return outp[:, :64]
